# baseline (speedup 1.0000x reference)
import functools

import jax
import jax.numpy as jnp
from jax import lax
from jax.experimental import pallas as pl
from jax.experimental.pallas import tpu as pltpu
from jax.experimental.pallas import tpu_sc as plsc

B = 128
N = 50
NP = 64
E = 500
EP = 512
IN = 256
MID = 64
MH = 4
OUT = 32
H = 4
ODIM = OUT * H
WDIM = N * ODIM
G = 4
CB = 640

_LANES = 16
_CSLOTS = NP * NP


def _count_sc_body(ei_hbm, zeros_hbm, c_hbm, src_v, dst_v, priv_v, out_v):
    cid = lax.axis_index("c")
    sid = lax.axis_index("s")

    @pl.when(jnp.logical_and(cid == 0, sid == 0))
    def _():
        pltpu.sync_copy(ei_hbm.at[0], src_v)
        pltpu.sync_copy(ei_hbm.at[1], dst_v)
        pltpu.sync_copy(zeros_hbm, priv_v)

        lane_base = lax.iota(jnp.int32, _LANES) * _CSLOTS
        ones = jnp.ones((_LANES,), jnp.float32)
        for ch in range(EP // _LANES):
            s = src_v[pl.ds(ch * _LANES, _LANES)]
            d = dst_v[pl.ds(ch * _LANES, _LANES)]
            idx = d * NP + s + lane_base
            plsc.addupdate_scatter(priv_v, [idx], ones)

        def red_body(j, carry):
            acc = jnp.zeros((_LANES,), jnp.float32)
            for l in range(_LANES):
                acc = acc + priv_v[pl.ds(l * _CSLOTS + j * _LANES, _LANES)]
            out_v[pl.ds(j * _LANES, _LANES)] = acc
            return carry

        lax.fori_loop(0, _CSLOTS // _LANES, red_body, 0)
        pltpu.sync_copy(out_v, c_hbm)


@functools.lru_cache(maxsize=1)
def _count_sc():
    return functools.partial(
        pl.kernel,
        out_type=jax.ShapeDtypeStruct((_CSLOTS,), jnp.float32),
        scratch_types=[
            pltpu.VMEM((EP,), jnp.int32),
            pltpu.VMEM((EP,), jnp.int32),
            pltpu.VMEM((_LANES * _CSLOTS,), jnp.float32),
            pltpu.VMEM((_CSLOTS,), jnp.float32),
        ],
        mesh=plsc.VectorSubcoreMesh(core_axis_name="c", subcore_axis_name="s"),
        compiler_params=pltpu.CompilerParams(needs_layout_passes=False),
    )(_count_sc_body)


def _leaky(x, slope):
    return jnp.where(x >= 0, x, slope * x)


def _attn_layer(hg, Ct, Cpos, BD, SEL, ALT, AR, heads, dh):
    elr = lax.dot_general(ALT, hg, (((1,), (1,)), ((), ())),
                          preferred_element_type=jnp.float32)
    elr_tiled = jnp.concatenate([elr] * heads, axis=1) * SEL
    elr_rep = jnp.dot(jnp.ones((NP, heads), jnp.float32), elr_tiled,
                      preferred_element_type=jnp.float32)
    er = jnp.dot(hg, AR, preferred_element_type=jnp.float32)
    er_rep = jnp.dot(er, SEL, preferred_element_type=jnp.float32)
    e = _leaky(er_rep + elr_rep, 0.2)
    em = jnp.where(Cpos, e, -1e30)
    rmax = jnp.max(em, axis=1, keepdims=True)
    ee = Ct * jnp.exp(em - rmax)
    den = jnp.dot(ee, BD, preferred_element_type=jnp.float32)
    alpha = ee / (den + 1e-9)
    outs = []
    for h in range(heads):
        outs.append(jnp.dot(alpha[:, h * NP:(h + 1) * NP],
                            hg[:, h * dh:(h + 1) * dh],
                            preferred_element_type=jnp.float32))
    return jnp.concatenate(outs, axis=1)


def _gat_tc(x_ref, W0_ref, AL0T_ref, AR0_ref, b0_ref, W1_ref, AL1T_ref,
            AR1_ref, b1_ref, WmapT_ref, bmap_ref, Ct_ref, BD_ref, SEL_ref,
            gf_ref, m_ref):
    Ct = Ct_ref[...]
    Cpos = Ct > 0.0
    BD = BD_ref[...]
    SEL = SEL_ref[...]
    x = x_ref[...]
    h0 = jnp.dot(x, W0_ref[...], preferred_element_type=jnp.float32)
    m_ref[...] = jnp.dot(x, WmapT_ref[...],
                         preferred_element_type=jnp.float32) + bmap_ref[...]
    for g in range(G):
        h0g = h0[g * NP:(g + 1) * NP, :]
        x1 = _attn_layer(h0g, Ct, Cpos, BD, SEL, AL0T_ref[...], AR0_ref[...],
                         MH, MID)
        x1 = _leaky(x1 + b0_ref[...], 0.1)
        h1 = jnp.dot(x1, W1_ref[...], preferred_element_type=jnp.float32)
        gfg = _attn_layer(h1, Ct, Cpos, BD, SEL, AL1T_ref[...], AR1_ref[...],
                          H, OUT)
        gf_ref[g * NP:(g + 1) * NP, :] = gfg + b1_ref[...]


def _gate_tc(Ww_ref, gfT_ref, gfTb_ref, mTb_ref, bwT_ref, outT_ref):
    logits = jnp.dot(Ww_ref[...], gfT_ref[...],
                     preferred_element_type=jnp.float32) + bwT_ref[...]
    w = jax.nn.sigmoid(logits)
    outT_ref[...] = (1.0 - w) * gfTb_ref[...] + w * mTb_ref[...]


def kernel(node_feature, edge_index, W0, al0, ar0, b0, W1, al1, ar1, b1,
           Wmap, bmap, Ww, bw):
    ei_pad = jnp.concatenate(
        [edge_index, jnp.full((2, EP - E), NP - 1, dtype=edge_index.dtype)],
        axis=1)

    C_flat = _count_sc()(ei_pad, jnp.zeros((_LANES * _CSLOTS,), jnp.float32))
    C = C_flat.reshape(NP, NP)

    xp = jnp.pad(node_feature, ((0, 0), (0, NP - N), (0, 0)))
    xp = xp.reshape(B * NP, IN)

    sel0 = jnp.repeat(jnp.eye(MH, dtype=jnp.float32), MID, axis=1)
    sel1 = jnp.repeat(jnp.eye(H, dtype=jnp.float32), OUT, axis=1)
    AL0T = al0.reshape(1, MH * MID) * sel0
    AR0 = (ar0.reshape(1, MH * MID) * sel0).T
    AL1T = al1.reshape(1, H * OUT) * sel1
    AR1 = (ar1.reshape(1, H * OUT) * sel1).T
    Ct = jnp.tile(C, (1, MH))
    eyeh = jnp.eye(MH, dtype=jnp.float32)
    SEL = jnp.repeat(eyeh, NP, axis=1)
    BD = jnp.kron(eyeh, jnp.ones((NP, NP), jnp.float32))

    gf_m = pl.pallas_call(
        _gat_tc,
        grid=(B // G,),
        in_specs=[
            pl.BlockSpec((G * NP, IN), lambda i: (i, 0)),
            pl.BlockSpec((IN, MH * MID), lambda i: (0, 0)),
            pl.BlockSpec((MH, MH * MID), lambda i: (0, 0)),
            pl.BlockSpec((MH * MID, MH), lambda i: (0, 0)),
            pl.BlockSpec((1, MH * MID), lambda i: (0, 0)),
            pl.BlockSpec((MH * MID, H * OUT), lambda i: (0, 0)),
            pl.BlockSpec((H, H * OUT), lambda i: (0, 0)),
            pl.BlockSpec((H * OUT, H), lambda i: (0, 0)),
            pl.BlockSpec((1, H * OUT), lambda i: (0, 0)),
            pl.BlockSpec((IN, ODIM), lambda i: (0, 0)),
            pl.BlockSpec((1, ODIM), lambda i: (0, 0)),
            pl.BlockSpec((NP, MH * NP), lambda i: (0, 0)),
            pl.BlockSpec((MH * NP, MH * NP), lambda i: (0, 0)),
            pl.BlockSpec((MH, MH * NP), lambda i: (0, 0)),
        ],
        out_specs=[
            pl.BlockSpec((G * NP, ODIM), lambda i: (i, 0)),
            pl.BlockSpec((G * NP, ODIM), lambda i: (i, 0)),
        ],
        out_shape=[
            jax.ShapeDtypeStruct((B * NP, ODIM), jnp.float32),
            jax.ShapeDtypeStruct((B * NP, ODIM), jnp.float32),
        ],
    )(xp, W0, AL0T, AR0, b0.reshape(1, -1), W1, AL1T, AR1, b1.reshape(1, -1),
      Wmap.T, bmap.reshape(1, -1), Ct, BD, SEL)
    gf, m = gf_m

    gf = gf.reshape(B, NP, ODIM)[:, :N, :]
    m = m.reshape(B, NP, ODIM)[:, :N, :]
    gfT = gf.reshape(B, WDIM).T
    mT = m.reshape(B, WDIM).T

    outT = pl.pallas_call(
        _gate_tc,
        grid=(WDIM // CB,),
        in_specs=[
            pl.BlockSpec((CB, WDIM), lambda k: (k, 0)),
            pl.BlockSpec((WDIM, B), lambda k: (0, 0)),
            pl.BlockSpec((CB, B), lambda k: (k, 0)),
            pl.BlockSpec((CB, B), lambda k: (k, 0)),
            pl.BlockSpec((CB, 1), lambda k: (k, 0)),
        ],
        out_specs=pl.BlockSpec((CB, B), lambda k: (k, 0)),
        out_shape=jax.ShapeDtypeStruct((WDIM, B), jnp.float32),
    )(Ww, gfT, gfT, mT, bw.reshape(WDIM, 1))

    return outT.T.reshape(B, N, ODIM)


# speedup vs baseline: 1.2222x; 1.2222x over previous
import functools

import jax
import jax.numpy as jnp
from jax import lax
from jax.experimental import pallas as pl
from jax.experimental.pallas import tpu as pltpu
from jax.experimental.pallas import tpu_sc as plsc

B = 128
N = 50
NP = 64
E = 500
EP = 512
IN = 256
MID = 64
MH = 4
OUT = 32
H = 4
ODIM = OUT * H
WDIM = N * ODIM
G = 4
CB = 640

_LANES = 16
_CSLOTS = NP * NP


def _count_sc_body(ei_hbm, zeros_hbm, c_hbm, src_v, dst_v, priv_v, out_v):
    cid = lax.axis_index("c")
    sid = lax.axis_index("s")

    @pl.when(jnp.logical_and(cid == 0, sid == 0))
    def _():
        pltpu.sync_copy(ei_hbm.at[0], src_v)
        pltpu.sync_copy(ei_hbm.at[1], dst_v)
        pltpu.sync_copy(zeros_hbm, priv_v)

        lane_base = lax.iota(jnp.int32, _LANES) * _CSLOTS
        ones = jnp.ones((_LANES,), jnp.float32)
        for ch in range(EP // _LANES):
            s = src_v[pl.ds(ch * _LANES, _LANES)]
            d = dst_v[pl.ds(ch * _LANES, _LANES)]
            idx = d * NP + s + lane_base
            plsc.addupdate_scatter(priv_v, [idx], ones)

        def red_body(j, carry):
            acc = jnp.zeros((_LANES,), jnp.float32)
            for l in range(_LANES):
                acc = acc + priv_v[pl.ds(l * _CSLOTS + j * _LANES, _LANES)]
            out_v[pl.ds(j * _LANES, _LANES)] = acc
            return carry

        lax.fori_loop(0, _CSLOTS // _LANES, red_body, 0)
        pltpu.sync_copy(out_v, c_hbm)


@functools.lru_cache(maxsize=1)
def _count_sc():
    return functools.partial(
        pl.kernel,
        out_type=jax.ShapeDtypeStruct((_CSLOTS,), jnp.float32),
        scratch_types=[
            pltpu.VMEM((EP,), jnp.int32),
            pltpu.VMEM((EP,), jnp.int32),
            pltpu.VMEM((_LANES * _CSLOTS,), jnp.float32),
            pltpu.VMEM((_CSLOTS,), jnp.float32),
        ],
        mesh=plsc.VectorSubcoreMesh(core_axis_name="c", subcore_axis_name="s"),
        compiler_params=pltpu.CompilerParams(needs_layout_passes=False),
    )(_count_sc_body)


def _leaky(x, slope):
    return jnp.where(x >= 0, x, slope * x)


def _attn_layer(hg, C, Cpos, ALT, AR, heads, dh):
    elr = lax.dot_general(ALT, hg, (((1,), (1,)), ((), ())),
                          preferred_element_type=jnp.float32)
    er = jnp.dot(hg, AR, preferred_element_type=jnp.float32)
    outs = []
    for h in range(heads):
        e = _leaky(elr[h:h + 1, :] + er[:, h:h + 1], 0.2)
        em = jnp.where(Cpos, e, -1e30)
        emax = jnp.max(em, axis=1, keepdims=True)
        ee = C * jnp.exp(em - emax)
        den = jnp.sum(ee, axis=1, keepdims=True)
        alpha = ee / (den + 1e-9)
        outs.append(jnp.dot(alpha, hg[:, h * dh:(h + 1) * dh],
                            preferred_element_type=jnp.float32))
    return jnp.concatenate(outs, axis=1)


def _gat_tc(x_ref, W0_ref, AL0T_ref, AR0_ref, b0_ref, W1_ref, AL1T_ref,
            AR1_ref, b1_ref, WmapT_ref, bmap_ref, C_ref, gf_ref, m_ref):
    C = C_ref[...]
    Cpos = C > 0.0
    x = x_ref[...]
    h0 = jnp.dot(x, W0_ref[...], preferred_element_type=jnp.float32)
    m_ref[...] = jnp.dot(x, WmapT_ref[...],
                         preferred_element_type=jnp.float32) + bmap_ref[...]
    for g in range(G):
        h0g = h0[g * NP:(g + 1) * NP, :]
        x1 = _attn_layer(h0g, C, Cpos, AL0T_ref[...], AR0_ref[...], MH, MID)
        x1 = _leaky(x1 + b0_ref[...], 0.1)
        h1 = jnp.dot(x1, W1_ref[...], preferred_element_type=jnp.float32)
        gfg = _attn_layer(h1, C, Cpos, AL1T_ref[...], AR1_ref[...], H, OUT)
        gf_ref[g * NP:(g + 1) * NP, :] = gfg + b1_ref[...]


def _gate_tc(Ww_ref, gfT_ref, gfTb_ref, mTb_ref, bwT_ref, outT_ref):
    logits = jnp.dot(Ww_ref[...], gfT_ref[...],
                     preferred_element_type=jnp.float32) + bwT_ref[...]
    w = jax.nn.sigmoid(logits)
    outT_ref[...] = (1.0 - w) * gfTb_ref[...] + w * mTb_ref[...]


def kernel(node_feature, edge_index, W0, al0, ar0, b0, W1, al1, ar1, b1,
           Wmap, bmap, Ww, bw):
    ei_pad = jnp.concatenate(
        [edge_index, jnp.full((2, EP - E), NP - 1, dtype=edge_index.dtype)],
        axis=1)

    C_flat = _count_sc()(ei_pad, jnp.zeros((_LANES * _CSLOTS,), jnp.float32))
    C = C_flat.reshape(NP, NP)

    xp = jnp.pad(node_feature, ((0, 0), (0, NP - N), (0, 0)))
    xp = xp.reshape(B * NP, IN)

    sel0 = jnp.repeat(jnp.eye(MH, dtype=jnp.float32), MID, axis=1)
    sel1 = jnp.repeat(jnp.eye(H, dtype=jnp.float32), OUT, axis=1)
    AL0T = al0.reshape(1, MH * MID) * sel0
    AR0 = (ar0.reshape(1, MH * MID) * sel0).T
    AL1T = al1.reshape(1, H * OUT) * sel1
    AR1 = (ar1.reshape(1, H * OUT) * sel1).T

    gf_m = pl.pallas_call(
        _gat_tc,
        grid=(B // G,),
        in_specs=[
            pl.BlockSpec((G * NP, IN), lambda i: (i, 0)),
            pl.BlockSpec((IN, MH * MID), lambda i: (0, 0)),
            pl.BlockSpec((MH, MH * MID), lambda i: (0, 0)),
            pl.BlockSpec((MH * MID, MH), lambda i: (0, 0)),
            pl.BlockSpec((1, MH * MID), lambda i: (0, 0)),
            pl.BlockSpec((MH * MID, H * OUT), lambda i: (0, 0)),
            pl.BlockSpec((H, H * OUT), lambda i: (0, 0)),
            pl.BlockSpec((H * OUT, H), lambda i: (0, 0)),
            pl.BlockSpec((1, H * OUT), lambda i: (0, 0)),
            pl.BlockSpec((IN, ODIM), lambda i: (0, 0)),
            pl.BlockSpec((1, ODIM), lambda i: (0, 0)),
            pl.BlockSpec((NP, NP), lambda i: (0, 0)),
        ],
        out_specs=[
            pl.BlockSpec((G * NP, ODIM), lambda i: (i, 0)),
            pl.BlockSpec((G * NP, ODIM), lambda i: (i, 0)),
        ],
        out_shape=[
            jax.ShapeDtypeStruct((B * NP, ODIM), jnp.float32),
            jax.ShapeDtypeStruct((B * NP, ODIM), jnp.float32),
        ],
    )(xp, W0, AL0T, AR0, b0.reshape(1, -1), W1, AL1T, AR1, b1.reshape(1, -1),
      Wmap.T, bmap.reshape(1, -1), C)
    gf, m = gf_m

    gf = gf.reshape(B, NP, ODIM)[:, :N, :]
    m = m.reshape(B, NP, ODIM)[:, :N, :]
    gfT = gf.reshape(B, WDIM).T
    mT = m.reshape(B, WDIM).T

    outT = pl.pallas_call(
        _gate_tc,
        grid=(WDIM // CB,),
        in_specs=[
            pl.BlockSpec((CB, WDIM), lambda k: (k, 0)),
            pl.BlockSpec((WDIM, B), lambda k: (0, 0)),
            pl.BlockSpec((CB, B), lambda k: (k, 0)),
            pl.BlockSpec((CB, B), lambda k: (k, 0)),
            pl.BlockSpec((CB, 1), lambda k: (k, 0)),
        ],
        out_specs=pl.BlockSpec((CB, B), lambda k: (k, 0)),
        out_shape=jax.ShapeDtypeStruct((WDIM, B), jnp.float32),
    )(Ww, gfT, gfT, mT, bw.reshape(WDIM, 1))

    return outT.T.reshape(B, N, ODIM)
